# Initial kernel scaffold; baseline (speedup 1.0000x reference)
import jax
import jax.numpy as jnp
from jax.experimental import pallas as pl


def kernel(Lap, x, W1, b1, W2, b2, tg, Wg, bg, tl, Wl, bl, Wp1, bp1, Wp2, bp2):
    raise NotImplementedError("write your pallas kernel here")


# speedup vs baseline: 2.0122x; 2.0122x over previous
import jax
import jax.numpy as jnp
from jax.experimental import pallas as pl
from jax.experimental.pallas import tpu as pltpu

N = 4096
IN = 256
H1 = 256
H2 = 128
TAU = 0.2

RB = 512
NBLK = N // RB


def _dotT(a, b):
    return jax.lax.dot_general(a, b, (((1,), (1,)), ((), ())),
                               preferred_element_type=jnp.float32)


def _dot(a, b):
    return jax.lax.dot_general(a, b, (((1,), (0,)), ((), ())),
                               preferred_element_type=jnp.float32)


def _pass1_body(x_ref, Lap_ref, W1_ref, b1_ref, W2_ref, b2_ref,
                Wg_ref, bg_ref, tg_ref, Wl_ref, bl_ref, tl_ref,
                yg_ref, yl_ref, h_ref):
    i = pl.program_id(0)

    @pl.when(i == 0)
    def _encode():
        h1 = jnp.maximum(_dotT(x_ref[...], W1_ref[...]) + b1_ref[...], 0.0)
        h2 = jnp.maximum(_dotT(h1, W2_ref[...]) + b2_ref[...], 0.0)
        h_ref[...] = h2

    h = h_ref[...]
    LX = _dot(Lap_ref[...], h)
    rows = h_ref[pl.ds(i * RB, RB), :]
    zg = rows - tg_ref[0, 0] * LX
    yg_ref[...] = jnp.maximum(_dotT(zg, Wg_ref[...]) + bg_ref[...], 0.0)
    zl = rows - tl_ref[...] * LX
    yl_ref[...] = jnp.maximum(_dotT(zl, Wl_ref[...]) + bl_ref[...], 0.0)


def _pass1(x, Lap, W1, b1, W2, b2, Wg0, bg0, tg0, Wl0, bl0, tl0):
    full = lambda r, c: pl.BlockSpec((r, c), lambda i: (0, 0))
    return pl.pallas_call(
        _pass1_body,
        grid=(NBLK,),
        in_specs=[
            full(N, IN),
            pl.BlockSpec((RB, N), lambda i: (i, 0)),
            full(H1, IN), full(1, H1),
            full(H2, H1), full(1, H2),
            full(H2, H2), full(1, H2), full(1, 1),
            full(H2, H2), full(1, H2), full(1, H2),
        ],
        out_specs=[pl.BlockSpec((RB, H2), lambda i: (i, 0))] * 2,
        out_shape=[jax.ShapeDtypeStruct((N, H2), jnp.float32)] * 2,
        scratch_shapes=[pltpu.VMEM((N, H2), jnp.float32)],
    )(x, Lap, W1, b1, W2, b2, Wg0, bg0, tg0, Wl0, bl0, tl0)


def _pass2_body(Lap_ref, xg_ref, xl_ref,
                Wg_ref, bg_ref, tg_ref, Wl_ref, bl_ref, tl_ref,
                yg_ref, yl_ref):
    i = pl.program_id(0)
    Lap = Lap_ref[...]
    LXg = _dot(Lap, xg_ref[...])
    LXl = _dot(Lap, xl_ref[...])
    zg = xg_ref[pl.ds(i * RB, RB), :] - tg_ref[0, 0] * LXg
    yg_ref[...] = jnp.maximum(_dotT(zg, Wg_ref[...]) + bg_ref[...], 0.0)
    zl = xl_ref[pl.ds(i * RB, RB), :] - tl_ref[...] * LXl
    yl_ref[...] = jnp.maximum(_dotT(zl, Wl_ref[...]) + bl_ref[...], 0.0)


def _pass2(Lap, xg, xl, Wg1, bg1, tg1, Wl1, bl1, tl1):
    full = lambda r, c: pl.BlockSpec((r, c), lambda i: (0, 0))
    return pl.pallas_call(
        _pass2_body,
        grid=(NBLK,),
        in_specs=[
            pl.BlockSpec((RB, N), lambda i: (i, 0)),
            full(N, H2), full(N, H2),
            full(H2, H2), full(1, H2), full(1, 1),
            full(H2, H2), full(1, H2), full(1, H2),
        ],
        out_specs=[pl.BlockSpec((RB, H2), lambda i: (i, 0))] * 2,
        out_shape=[jax.ShapeDtypeStruct((N, H2), jnp.float32)] * 2,
    )(Lap, xg, xl, Wg1, bg1, tg1, Wl1, bl1, tl1)


def _loss_body(xg_ref, xl_ref, Wp1_ref, bp1_ref, Wp2_ref, bp2_ref,
               out_ref, A_ref, B_ref, c_ref, acc_ref):
    i = pl.program_id(0)

    @pl.when(i == 0)
    def _project():
        zg = _dotT(xg_ref[...], Wp1_ref[...]) + bp1_ref[...]
        ng = jnp.sqrt(jnp.sum(zg * zg, axis=1, keepdims=True))
        A_ref[...] = zg / jnp.maximum(ng, 1e-12)
        zl = _dotT(xl_ref[...], Wp2_ref[...]) + bp2_ref[...]
        nl = jnp.sqrt(jnp.sum(zl * zl, axis=1, keepdims=True))
        B_ref[...] = zl / jnp.maximum(nl, 1e-12)
        acc_ref[0] = 0.0
        acc_ref[1] = 0.0

    A = A_ref[...]
    B = B_ref[...]
    A_blk = A_ref[pl.ds(i * RB, RB), :]
    B_blk = B_ref[pl.ds(i * RB, RB), :]

    row_id = jax.lax.broadcasted_iota(jnp.int32, (RB, N), 0) + i * RB
    col_id = jax.lax.broadcasted_iota(jnp.int32, (RB, N), 1)
    diag = row_id == col_id

    S_ab = _dotT(A_blk, B) * (1.0 / TAU)
    E_ab = jnp.where(diag, 0.0, jnp.exp(S_ab))
    r_ab = jnp.sum(E_ab, axis=1)
    d_ab = jnp.sum(jnp.where(diag, S_ab, 0.0), axis=1)
    c_ref[pl.ds(i, 1), :] = jnp.sum(E_ab, axis=0, keepdims=True)

    S_aa = _dotT(A_blk, A) * (1.0 / TAU)
    r_aa = jnp.sum(jnp.where(diag, 0.0, jnp.exp(S_aa)), axis=1)
    S_bb = _dotT(B_blk, B) * (1.0 / TAU)
    r_bb = jnp.sum(jnp.where(diag, 0.0, jnp.exp(S_bb)), axis=1)

    acc_ref[0] += jnp.sum(d_ab - jnp.log(r_ab) - jnp.log(r_aa))
    acc_ref[1] += jnp.sum(d_ab - jnp.log(r_bb))

    @pl.when(i == NBLK - 1)
    def _finish():
        csum = jnp.sum(c_ref[...], axis=0)
        loss1 = acc_ref[1] - jnp.sum(jnp.log(csum))
        out_ref[0, 0] = -(acc_ref[0] + loss1) / (2.0 * N)


def _loss(xg, xl, Wp1, bp1, Wp2, bp2):
    full = lambda r, c: pl.BlockSpec((r, c), lambda i: (0, 0))
    return pl.pallas_call(
        _loss_body,
        grid=(NBLK,),
        in_specs=[
            full(N, H2), full(N, H2),
            full(H2, H2), full(1, H2),
            full(H2, H2), full(1, H2),
        ],
        out_specs=pl.BlockSpec(memory_space=pltpu.SMEM),
        out_shape=jax.ShapeDtypeStruct((1, 1), jnp.float32),
        scratch_shapes=[
            pltpu.VMEM((N, H2), jnp.float32),
            pltpu.VMEM((N, H2), jnp.float32),
            pltpu.VMEM((NBLK, N), jnp.float32),
            pltpu.SMEM((2,), jnp.float32),
        ],
    )(xg, xl, Wp1, bp1, Wp2, bp2)


def kernel(Lap, x, W1, b1, W2, b2, tg, Wg, bg, tl, Wl, bl, Wp1, bp1, Wp2, bp2):
    r1 = lambda v: v.reshape(1, -1)
    xg1, xl1 = _pass1(x, Lap, W1, r1(b1), W2, r1(b2),
                      Wg[0], r1(bg[0]), tg[0].reshape(1, 1),
                      Wl[0], r1(bl[0]), r1(tl[0]))
    xg2, xl2 = _pass2(Lap, xg1, xl1,
                      Wg[1], r1(bg[1]), tg[1].reshape(1, 1),
                      Wl[1], r1(bl[1]), r1(tl[1]))
    loss = _loss(xg2, xl2, Wp1, r1(bp1), Wp2, r1(bp2))
    return (xg2, xl2, loss[0, 0])


# speedup vs baseline: 2.0297x; 1.0087x over previous
import jax
import jax.numpy as jnp
from jax.experimental import pallas as pl
from jax.experimental.pallas import tpu as pltpu

N = 4096
IN = 256
H1 = 256
H2 = 128
TAU = 0.2

RB = 512
NBLK = N // RB


def _dotT(a, b):
    return jax.lax.dot_general(a, b, (((1,), (1,)), ((), ())),
                               preferred_element_type=jnp.float32)


def _dot(a, b):
    return jax.lax.dot_general(a, b, (((1,), (0,)), ((), ())),
                               preferred_element_type=jnp.float32)


def _dot16(a, b):
    return jax.lax.dot_general(a.astype(jnp.bfloat16), b.astype(jnp.bfloat16),
                               (((1,), (0,)), ((), ())),
                               preferred_element_type=jnp.float32)


def _dotT16(a, b):
    return jax.lax.dot_general(a.astype(jnp.bfloat16), b.astype(jnp.bfloat16),
                               (((1,), (1,)), ((), ())),
                               preferred_element_type=jnp.float32)


def _pass1_body(x_ref, Lap_ref, W1_ref, b1_ref, W2_ref, b2_ref,
                Wg_ref, bg_ref, tg_ref, Wl_ref, bl_ref, tl_ref,
                yg_ref, yl_ref, h_ref):
    i = pl.program_id(0)

    @pl.when(i == 0)
    def _encode():
        h1 = jnp.maximum(_dotT(x_ref[...], W1_ref[...]) + b1_ref[...], 0.0)
        h2 = jnp.maximum(_dotT(h1, W2_ref[...]) + b2_ref[...], 0.0)
        h_ref[...] = h2

    h = h_ref[...]
    LX = _dot16(Lap_ref[...], h)
    rows = h_ref[pl.ds(i * RB, RB), :]
    zg = rows - tg_ref[0, 0] * LX
    yg_ref[...] = jnp.maximum(_dotT(zg, Wg_ref[...]) + bg_ref[...], 0.0)
    zl = rows - tl_ref[...] * LX
    yl_ref[...] = jnp.maximum(_dotT(zl, Wl_ref[...]) + bl_ref[...], 0.0)


def _pass1(x, Lap, W1, b1, W2, b2, Wg0, bg0, tg0, Wl0, bl0, tl0):
    full = lambda r, c: pl.BlockSpec((r, c), lambda i: (0, 0))
    return pl.pallas_call(
        _pass1_body,
        grid=(NBLK,),
        in_specs=[
            full(N, IN),
            pl.BlockSpec((RB, N), lambda i: (i, 0)),
            full(H1, IN), full(1, H1),
            full(H2, H1), full(1, H2),
            full(H2, H2), full(1, H2), full(1, 1),
            full(H2, H2), full(1, H2), full(1, H2),
        ],
        out_specs=[pl.BlockSpec((RB, H2), lambda i: (i, 0))] * 2,
        out_shape=[jax.ShapeDtypeStruct((N, H2), jnp.float32)] * 2,
        scratch_shapes=[pltpu.VMEM((N, H2), jnp.float32)],
    )(x, Lap, W1, b1, W2, b2, Wg0, bg0, tg0, Wl0, bl0, tl0)


def _pass2_body(Lap_ref, xg_ref, xl_ref,
                Wg_ref, bg_ref, tg_ref, Wl_ref, bl_ref, tl_ref,
                yg_ref, yl_ref):
    i = pl.program_id(0)
    Lap = Lap_ref[...].astype(jnp.bfloat16)
    LXg = _dot16(Lap, xg_ref[...])
    LXl = _dot16(Lap, xl_ref[...])
    zg = xg_ref[pl.ds(i * RB, RB), :] - tg_ref[0, 0] * LXg
    yg_ref[...] = jnp.maximum(_dotT(zg, Wg_ref[...]) + bg_ref[...], 0.0)
    zl = xl_ref[pl.ds(i * RB, RB), :] - tl_ref[...] * LXl
    yl_ref[...] = jnp.maximum(_dotT(zl, Wl_ref[...]) + bl_ref[...], 0.0)


def _pass2(Lap, xg, xl, Wg1, bg1, tg1, Wl1, bl1, tl1):
    full = lambda r, c: pl.BlockSpec((r, c), lambda i: (0, 0))
    return pl.pallas_call(
        _pass2_body,
        grid=(NBLK,),
        in_specs=[
            pl.BlockSpec((RB, N), lambda i: (i, 0)),
            full(N, H2), full(N, H2),
            full(H2, H2), full(1, H2), full(1, 1),
            full(H2, H2), full(1, H2), full(1, H2),
        ],
        out_specs=[pl.BlockSpec((RB, H2), lambda i: (i, 0))] * 2,
        out_shape=[jax.ShapeDtypeStruct((N, H2), jnp.float32)] * 2,
    )(Lap, xg, xl, Wg1, bg1, tg1, Wl1, bl1, tl1)


def _loss_body(xg_ref, xl_ref, Wp1_ref, bp1_ref, Wp2_ref, bp2_ref,
               out_ref, A_ref, B_ref, c_ref, acc_ref):
    i = pl.program_id(0)

    @pl.when(i == 0)
    def _project():
        zg = _dotT(xg_ref[...], Wp1_ref[...]) + bp1_ref[...]
        ng = jnp.sqrt(jnp.sum(zg * zg, axis=1, keepdims=True))
        A_ref[...] = (zg / jnp.maximum(ng, 1e-12)).astype(jnp.bfloat16)
        zl = _dotT(xl_ref[...], Wp2_ref[...]) + bp2_ref[...]
        nl = jnp.sqrt(jnp.sum(zl * zl, axis=1, keepdims=True))
        B_ref[...] = (zl / jnp.maximum(nl, 1e-12)).astype(jnp.bfloat16)
        acc_ref[0] = 0.0
        acc_ref[1] = 0.0

    A = A_ref[...]
    B = B_ref[...]
    A_blk = A_ref[pl.ds(i * RB, RB), :]
    B_blk = B_ref[pl.ds(i * RB, RB), :]

    row_id = jax.lax.broadcasted_iota(jnp.int32, (RB, N), 0) + i * RB
    col_id = jax.lax.broadcasted_iota(jnp.int32, (RB, N), 1)
    diag = row_id == col_id

    S_ab = _dotT16(A_blk, B) * (1.0 / TAU)
    E_ab = jnp.where(diag, 0.0, jnp.exp(S_ab))
    r_ab = jnp.sum(E_ab, axis=1)
    d_ab = jnp.sum(jnp.where(diag, S_ab, 0.0), axis=1)
    c_ref[pl.ds(i, 1), :] = jnp.sum(E_ab, axis=0, keepdims=True)

    S_aa = _dotT16(A_blk, A) * (1.0 / TAU)
    r_aa = jnp.sum(jnp.where(diag, 0.0, jnp.exp(S_aa)), axis=1)
    S_bb = _dotT16(B_blk, B) * (1.0 / TAU)
    r_bb = jnp.sum(jnp.where(diag, 0.0, jnp.exp(S_bb)), axis=1)

    acc_ref[0] += jnp.sum(d_ab - jnp.log(r_ab) - jnp.log(r_aa))
    acc_ref[1] += jnp.sum(d_ab - jnp.log(r_bb))

    @pl.when(i == NBLK - 1)
    def _finish():
        csum = jnp.sum(c_ref[...], axis=0)
        loss1 = acc_ref[1] - jnp.sum(jnp.log(csum))
        out_ref[0, 0] = -(acc_ref[0] + loss1) / (2.0 * N)


def _loss(xg, xl, Wp1, bp1, Wp2, bp2):
    full = lambda r, c: pl.BlockSpec((r, c), lambda i: (0, 0))
    return pl.pallas_call(
        _loss_body,
        grid=(NBLK,),
        in_specs=[
            full(N, H2), full(N, H2),
            full(H2, H2), full(1, H2),
            full(H2, H2), full(1, H2),
        ],
        out_specs=pl.BlockSpec(memory_space=pltpu.SMEM),
        out_shape=jax.ShapeDtypeStruct((1, 1), jnp.float32),
        scratch_shapes=[
            pltpu.VMEM((N, H2), jnp.bfloat16),
            pltpu.VMEM((N, H2), jnp.bfloat16),
            pltpu.VMEM((NBLK, N), jnp.float32),
            pltpu.SMEM((2,), jnp.float32),
        ],
    )(xg, xl, Wp1, bp1, Wp2, bp2)


def kernel(Lap, x, W1, b1, W2, b2, tg, Wg, bg, tl, Wl, bl, Wp1, bp1, Wp2, bp2):
    r1 = lambda v: v.reshape(1, -1)
    xg1, xl1 = _pass1(x, Lap, W1, r1(b1), W2, r1(b2),
                      Wg[0], r1(bg[0]), tg[0].reshape(1, 1),
                      Wl[0], r1(bl[0]), r1(tl[0]))
    xg2, xl2 = _pass2(Lap, xg1, xl1,
                      Wg[1], r1(bg[1]), tg[1].reshape(1, 1),
                      Wl[1], r1(bl[1]), r1(tl[1]))
    loss = _loss(xg2, xl2, Wp1, r1(bp1), Wp2, r1(bp2))
    return (xg2, xl2, loss[0, 0])


# speedup vs baseline: 2.4726x; 1.2182x over previous
import jax
import jax.numpy as jnp
from jax.experimental import pallas as pl
from jax.experimental.pallas import tpu as pltpu

N = 4096
IN = 256
H1 = 256
H2 = 128
TAU = 0.2

RB = 512
NBLK = N // RB


def _dotT(a, b):
    return jax.lax.dot_general(a, b, (((1,), (1,)), ((), ())),
                               preferred_element_type=jnp.float32)


def _dot(a, b):
    return jax.lax.dot_general(a, b, (((1,), (0,)), ((), ())),
                               preferred_element_type=jnp.float32)


def _dot16(a, b):
    return jax.lax.dot_general(a.astype(jnp.bfloat16), b.astype(jnp.bfloat16),
                               (((1,), (0,)), ((), ())),
                               preferred_element_type=jnp.float32)


def _dotT16(a, b):
    return jax.lax.dot_general(a.astype(jnp.bfloat16), b.astype(jnp.bfloat16),
                               (((1,), (1,)), ((), ())),
                               preferred_element_type=jnp.float32)


def _pass1_body(x_ref, Lap_ref, W1_ref, b1_ref, W2_ref, b2_ref,
                Wg_ref, bg_ref, tg_ref, Wl_ref, bl_ref, tl_ref,
                yg_ref, yl_ref, h_ref):
    i = pl.program_id(0)

    @pl.when(i == 0)
    def _encode():
        h1 = jnp.maximum(_dotT(x_ref[...], W1_ref[...]) + b1_ref[...], 0.0)
        h2 = jnp.maximum(_dotT(h1, W2_ref[...]) + b2_ref[...], 0.0)
        h_ref[...] = h2

    h = h_ref[...]
    LX = _dot16(Lap_ref[...], h)
    rows = h_ref[pl.ds(i * RB, RB), :]
    zg = rows - tg_ref[0, 0] * LX
    yg_ref[...] = jnp.maximum(_dotT(zg, Wg_ref[...]) + bg_ref[...], 0.0)
    zl = rows - tl_ref[...] * LX
    yl_ref[...] = jnp.maximum(_dotT(zl, Wl_ref[...]) + bl_ref[...], 0.0)


def _pass1(x, Lap, W1, b1, W2, b2, Wg0, bg0, tg0, Wl0, bl0, tl0):
    full = lambda r, c: pl.BlockSpec((r, c), lambda i: (0, 0))
    return pl.pallas_call(
        _pass1_body,
        grid=(NBLK,),
        in_specs=[
            full(N, IN),
            pl.BlockSpec((RB, N), lambda i: (i, 0)),
            full(H1, IN), full(1, H1),
            full(H2, H1), full(1, H2),
            full(H2, H2), full(1, H2), full(1, 1),
            full(H2, H2), full(1, H2), full(1, H2),
        ],
        out_specs=[pl.BlockSpec((RB, H2), lambda i: (i, 0))] * 2,
        out_shape=[jax.ShapeDtypeStruct((N, H2), jnp.float32)] * 2,
        scratch_shapes=[pltpu.VMEM((N, H2), jnp.float32)],
    )(x, Lap, W1, b1, W2, b2, Wg0, bg0, tg0, Wl0, bl0, tl0)


def _pass2_body(Lap_ref, xg_ref, xl_ref,
                Wg_ref, bg_ref, tg_ref, Wl_ref, bl_ref, tl_ref,
                yg_ref, yl_ref):
    i = pl.program_id(0)
    Lap = Lap_ref[...].astype(jnp.bfloat16)
    LXg = _dot16(Lap, xg_ref[...])
    LXl = _dot16(Lap, xl_ref[...])
    zg = xg_ref[pl.ds(i * RB, RB), :] - tg_ref[0, 0] * LXg
    yg_ref[...] = jnp.maximum(_dotT(zg, Wg_ref[...]) + bg_ref[...], 0.0)
    zl = xl_ref[pl.ds(i * RB, RB), :] - tl_ref[...] * LXl
    yl_ref[...] = jnp.maximum(_dotT(zl, Wl_ref[...]) + bl_ref[...], 0.0)


def _pass2(Lap, xg, xl, Wg1, bg1, tg1, Wl1, bl1, tl1):
    full = lambda r, c: pl.BlockSpec((r, c), lambda i: (0, 0))
    return pl.pallas_call(
        _pass2_body,
        grid=(NBLK,),
        in_specs=[
            pl.BlockSpec((RB, N), lambda i: (i, 0)),
            full(N, H2), full(N, H2),
            full(H2, H2), full(1, H2), full(1, 1),
            full(H2, H2), full(1, H2), full(1, H2),
        ],
        out_specs=[pl.BlockSpec((RB, H2), lambda i: (i, 0))] * 2,
        out_shape=[jax.ShapeDtypeStruct((N, H2), jnp.float32)] * 2,
    )(Lap, xg, xl, Wg1, bg1, tg1, Wl1, bl1, tl1)


def _loss_body(xg_ref, xl_ref, Wp1_ref, bp1_ref, Wp2_ref, bp2_ref,
               out_ref, A_ref, B_ref, c_ref, e_ref, acc_ref):
    i = pl.program_id(0)
    isq = 1.0 / (TAU ** 0.5)

    @pl.when(i == 0)
    def _project():
        zg = _dotT(xg_ref[...], Wp1_ref[...]) + bp1_ref[...]
        ng = jnp.sqrt(jnp.sum(zg * zg, axis=1, keepdims=True))
        A_ref[...] = (zg * (isq / jnp.maximum(ng, 1e-12))).astype(jnp.bfloat16)
        zl = _dotT(xl_ref[...], Wp2_ref[...]) + bp2_ref[...]
        nl = jnp.sqrt(jnp.sum(zl * zl, axis=1, keepdims=True))
        B_ref[...] = (zl * (isq / jnp.maximum(nl, 1e-12))).astype(jnp.bfloat16)
        acc_ref[0] = 0.0
        acc_ref[1] = 0.0

    A = A_ref[...]
    B = B_ref[...]
    A_blk = A_ref[pl.ds(i * RB, RB), :]
    B_blk = B_ref[pl.ds(i * RB, RB), :]
    Af = A_blk.astype(jnp.float32)
    Bf = B_blk.astype(jnp.float32)

    d_ab = jnp.sum(Af * Bf, axis=1, keepdims=True)
    e_ab = jnp.exp(d_ab)
    E_ab = jnp.exp(_dotT16(A_blk, B))
    r_ab = jnp.sum(E_ab, axis=1, keepdims=True) - e_ab
    c_ref[pl.ds(i, 1), :] = jnp.sum(E_ab, axis=0, keepdims=True)
    e_ref[pl.ds(i * RB, RB), :] = e_ab

    r_aa = (jnp.sum(jnp.exp(_dotT16(A_blk, A)), axis=1, keepdims=True)
            - jnp.exp(jnp.sum(Af * Af, axis=1, keepdims=True)))
    r_bb = (jnp.sum(jnp.exp(_dotT16(B_blk, B)), axis=1, keepdims=True)
            - jnp.exp(jnp.sum(Bf * Bf, axis=1, keepdims=True)))

    acc_ref[0] += jnp.sum(d_ab - jnp.log(r_ab) - jnp.log(r_aa))
    acc_ref[1] += jnp.sum(d_ab - jnp.log(r_bb))

    @pl.when(i == NBLK - 1)
    def _finish():
        csum = jnp.sum(c_ref[...], axis=0) - e_ref[...].reshape(N)
        loss1 = acc_ref[1] - jnp.sum(jnp.log(csum))
        out_ref[0, 0] = -(acc_ref[0] + loss1) / (2.0 * N)


def _loss(xg, xl, Wp1, bp1, Wp2, bp2):
    full = lambda r, c: pl.BlockSpec((r, c), lambda i: (0, 0))
    return pl.pallas_call(
        _loss_body,
        grid=(NBLK,),
        in_specs=[
            full(N, H2), full(N, H2),
            full(H2, H2), full(1, H2),
            full(H2, H2), full(1, H2),
        ],
        out_specs=pl.BlockSpec(memory_space=pltpu.SMEM),
        out_shape=jax.ShapeDtypeStruct((1, 1), jnp.float32),
        scratch_shapes=[
            pltpu.VMEM((N, H2), jnp.bfloat16),
            pltpu.VMEM((N, H2), jnp.bfloat16),
            pltpu.VMEM((NBLK, N), jnp.float32),
            pltpu.VMEM((N, 1), jnp.float32),
            pltpu.SMEM((2,), jnp.float32),
        ],
    )(xg, xl, Wp1, bp1, Wp2, bp2)


def kernel(Lap, x, W1, b1, W2, b2, tg, Wg, bg, tl, Wl, bl, Wp1, bp1, Wp2, bp2):
    r1 = lambda v: v.reshape(1, -1)
    xg1, xl1 = _pass1(x, Lap, W1, r1(b1), W2, r1(b2),
                      Wg[0], r1(bg[0]), tg[0].reshape(1, 1),
                      Wl[0], r1(bl[0]), r1(tl[0]))
    xg2, xl2 = _pass2(Lap, xg1, xl1,
                      Wg[1], r1(bg[1]), tg[1].reshape(1, 1),
                      Wl[1], r1(bl[1]), r1(tl[1]))
    loss = _loss(xg2, xl2, Wp1, r1(bp1), Wp2, r1(bp2))
    return (xg2, xl2, loss[0, 0])


# speedup vs baseline: 2.6469x; 1.0705x over previous
import jax
import jax.numpy as jnp
from jax.experimental import pallas as pl
from jax.experimental.pallas import tpu as pltpu

N = 4096
IN = 256
H1 = 256
H2 = 128
TAU = 0.2

RB = 512
NBLK = N // RB


def _dotT(a, b):
    return jax.lax.dot_general(a, b, (((1,), (1,)), ((), ())),
                               preferred_element_type=jnp.float32)


def _dot16(a, b):
    return jax.lax.dot_general(a.astype(jnp.bfloat16), b.astype(jnp.bfloat16),
                               (((1,), (0,)), ((), ())),
                               preferred_element_type=jnp.float32)


def _dotT16(a, b):
    return jax.lax.dot_general(a.astype(jnp.bfloat16), b.astype(jnp.bfloat16),
                               (((1,), (1,)), ((), ())),
                               preferred_element_type=jnp.float32)


def _pass1_body(x_ref, Lap_ref, W1_ref, b1_ref, W2_ref, b2_ref,
                Wg_ref, bg_ref, tg_ref, Wl_ref, bl_ref, tl_ref,
                yg_ref, yl_ref, h_ref, h16_ref):
    i = pl.program_id(0)

    @pl.when(i == 0)
    def _encode():
        h1 = jnp.maximum(_dotT(x_ref[...], W1_ref[...]) + b1_ref[...], 0.0)
        h2 = jnp.maximum(_dotT(h1, W2_ref[...]) + b2_ref[...], 0.0)
        h_ref[...] = h2
        h16_ref[...] = h2.astype(jnp.bfloat16)

    LX = _dot16(Lap_ref[...], h16_ref[...])
    rows = h_ref[pl.ds(i * RB, RB), :]
    zg = rows - tg_ref[0] * LX
    yg_ref[...] = jnp.maximum(_dotT(zg, Wg_ref[0]) + bg_ref[0], 0.0)
    zl = rows - tl_ref[0] * LX
    yl_ref[...] = jnp.maximum(_dotT(zl, Wl_ref[0]) + bl_ref[0], 0.0)


def _pass1(x, Lap, W1, b1, W2, b2, tg, Wg, bg, tl, Wl, bl):
    full = lambda *dims: pl.BlockSpec(dims, lambda i: (0,) * len(dims))
    return pl.pallas_call(
        _pass1_body,
        grid=(NBLK,),
        in_specs=[
            full(N, IN),
            pl.BlockSpec((RB, N), lambda i: (i, 0)),
            full(H1, IN), full(1, H1),
            full(H2, H1), full(1, H2),
            pl.BlockSpec((1, H2, H2), lambda i: (0, 0, 0)),
            pl.BlockSpec((1, 1, H2), lambda i: (0, 0, 0)),
            pl.BlockSpec(memory_space=pltpu.SMEM),
            pl.BlockSpec((1, H2, H2), lambda i: (0, 0, 0)),
            pl.BlockSpec((1, 1, H2), lambda i: (0, 0, 0)),
            pl.BlockSpec((1, 1, H2), lambda i: (0, 0, 0)),
        ],
        out_specs=[pl.BlockSpec((RB, H2), lambda i: (i, 0))] * 2,
        out_shape=[jax.ShapeDtypeStruct((N, H2), jnp.float32)] * 2,
        scratch_shapes=[pltpu.VMEM((N, H2), jnp.float32),
                        pltpu.VMEM((N, H2), jnp.bfloat16)],
    )(x, Lap, W1, b1.reshape(1, H1), W2, b2.reshape(1, H2),
      Wg, bg.reshape(2, 1, H2), tg, Wl, bl.reshape(2, 1, H2), tl.reshape(2, 1, H2))


def _pass2_body(Lap_ref, xg_ref, xl_ref,
                Wg_ref, bg_ref, tg_ref, Wl_ref, bl_ref, tl_ref,
                yg_ref, yl_ref, g16_ref, l16_ref):
    i = pl.program_id(0)

    @pl.when(i == 0)
    def _cast():
        g16_ref[...] = xg_ref[...].astype(jnp.bfloat16)
        l16_ref[...] = xl_ref[...].astype(jnp.bfloat16)

    Lap = Lap_ref[...].astype(jnp.bfloat16)
    LXg = _dot16(Lap, g16_ref[...])
    LXl = _dot16(Lap, l16_ref[...])
    zg = xg_ref[pl.ds(i * RB, RB), :] - tg_ref[1] * LXg
    yg_ref[...] = jnp.maximum(_dotT(zg, Wg_ref[0]) + bg_ref[0], 0.0)
    zl = xl_ref[pl.ds(i * RB, RB), :] - tl_ref[0] * LXl
    yl_ref[...] = jnp.maximum(_dotT(zl, Wl_ref[0]) + bl_ref[0], 0.0)


def _pass2(Lap, xg, xl, tg, Wg, bg, tl, Wl, bl):
    full = lambda *dims: pl.BlockSpec(dims, lambda i: (0,) * len(dims))
    return pl.pallas_call(
        _pass2_body,
        grid=(NBLK,),
        in_specs=[
            pl.BlockSpec((RB, N), lambda i: (i, 0)),
            full(N, H2), full(N, H2),
            pl.BlockSpec((1, H2, H2), lambda i: (1, 0, 0)),
            pl.BlockSpec((1, 1, H2), lambda i: (1, 0, 0)),
            pl.BlockSpec(memory_space=pltpu.SMEM),
            pl.BlockSpec((1, H2, H2), lambda i: (1, 0, 0)),
            pl.BlockSpec((1, 1, H2), lambda i: (1, 0, 0)),
            pl.BlockSpec((1, 1, H2), lambda i: (1, 0, 0)),
        ],
        out_specs=[pl.BlockSpec((RB, H2), lambda i: (i, 0))] * 2,
        out_shape=[jax.ShapeDtypeStruct((N, H2), jnp.float32)] * 2,
        scratch_shapes=[pltpu.VMEM((N, H2), jnp.bfloat16),
                        pltpu.VMEM((N, H2), jnp.bfloat16)],
    )(Lap, xg, xl, Wg, bg.reshape(2, 1, H2), tg, Wl, bl.reshape(2, 1, H2), tl.reshape(2, 1, H2))


def _loss_body(xg_ref, xl_ref, Wp1_ref, bp1_ref, Wp2_ref, bp2_ref,
               out_ref, A_ref, B_ref, c_ref, e_ref, acc_ref):
    i = pl.program_id(0)
    isq = 1.0 / (TAU ** 0.5)

    @pl.when(i == 0)
    def _project():
        zg = _dotT16(xg_ref[...], Wp1_ref[...]) + bp1_ref[...]
        ng = jnp.sqrt(jnp.sum(zg * zg, axis=1, keepdims=True))
        A_ref[...] = (zg * (isq / jnp.maximum(ng, 1e-12))).astype(jnp.bfloat16)
        zl = _dotT16(xl_ref[...], Wp2_ref[...]) + bp2_ref[...]
        nl = jnp.sqrt(jnp.sum(zl * zl, axis=1, keepdims=True))
        B_ref[...] = (zl * (isq / jnp.maximum(nl, 1e-12))).astype(jnp.bfloat16)
        acc_ref[0] = 0.0
        acc_ref[1] = 0.0

    A = A_ref[...]
    B = B_ref[...]
    A_blk = A_ref[pl.ds(i * RB, RB), :]
    B_blk = B_ref[pl.ds(i * RB, RB), :]
    Af = A_blk.astype(jnp.float32)
    Bf = B_blk.astype(jnp.float32)

    d_ab = jnp.sum(Af * Bf, axis=1, keepdims=True)
    e_ab = jnp.exp(d_ab)
    E_ab = jnp.exp(_dotT16(A_blk, B))
    r_ab = jnp.sum(E_ab, axis=1, keepdims=True) - e_ab
    c_ref[pl.ds(i, 1), :] = jnp.sum(E_ab, axis=0, keepdims=True)
    e_ref[pl.ds(i * RB, RB), :] = e_ab

    r_aa = (jnp.sum(jnp.exp(_dotT16(A_blk, A)), axis=1, keepdims=True)
            - jnp.exp(jnp.sum(Af * Af, axis=1, keepdims=True)))
    r_bb = (jnp.sum(jnp.exp(_dotT16(B_blk, B)), axis=1, keepdims=True)
            - jnp.exp(jnp.sum(Bf * Bf, axis=1, keepdims=True)))

    acc_ref[0] += jnp.sum(d_ab - jnp.log(r_ab) - jnp.log(r_aa))
    acc_ref[1] += jnp.sum(d_ab - jnp.log(r_bb))

    @pl.when(i == NBLK - 1)
    def _finish():
        csum = jnp.sum(c_ref[...], axis=0) - e_ref[...].reshape(N)
        loss1 = acc_ref[1] - jnp.sum(jnp.log(csum))
        out_ref[0, 0] = -(acc_ref[0] + loss1) / (2.0 * N)


def _loss(xg, xl, Wp1, bp1, Wp2, bp2):
    full = lambda *dims: pl.BlockSpec(dims, lambda i: (0,) * len(dims))
    return pl.pallas_call(
        _loss_body,
        grid=(NBLK,),
        in_specs=[
            full(N, H2), full(N, H2),
            full(H2, H2), full(1, H2),
            full(H2, H2), full(1, H2),
        ],
        out_specs=pl.BlockSpec(memory_space=pltpu.SMEM),
        out_shape=jax.ShapeDtypeStruct((1, 1), jnp.float32),
        scratch_shapes=[
            pltpu.VMEM((N, H2), jnp.bfloat16),
            pltpu.VMEM((N, H2), jnp.bfloat16),
            pltpu.VMEM((NBLK, N), jnp.float32),
            pltpu.VMEM((N, 1), jnp.float32),
            pltpu.SMEM((2,), jnp.float32),
        ],
    )(xg, xl, Wp1, bp1.reshape(1, H2), Wp2, bp2.reshape(1, H2))


def kernel(Lap, x, W1, b1, W2, b2, tg, Wg, bg, tl, Wl, bl, Wp1, bp1, Wp2, bp2):
    xg1, xl1 = _pass1(x, Lap, W1, b1, W2, b2, tg, Wg, bg, tl, Wl, bl)
    xg2, xl2 = _pass2(Lap, xg1, xl1, tg, Wg, bg, tl, Wl, bl)
    loss = _loss(xg2, xl2, Wp1, bp1, Wp2, bp2)
    return (xg2, xl2, loss[0, 0])


# speedup vs baseline: 2.6600x; 1.0049x over previous
import jax
import jax.numpy as jnp
from jax.experimental import pallas as pl
from jax.experimental.pallas import tpu as pltpu

N = 4096
IN = 256
H1 = 256
H2 = 128
TAU = 0.2

RB = 1024
NBLK = N // RB
RL = 512
NL = N // RL


def _dotT(a, b):
    return jax.lax.dot_general(a, b, (((1,), (1,)), ((), ())),
                               preferred_element_type=jnp.float32)


def _dot16(a, b):
    return jax.lax.dot_general(a.astype(jnp.bfloat16), b.astype(jnp.bfloat16),
                               (((1,), (0,)), ((), ())),
                               preferred_element_type=jnp.float32)


def _dotT16(a, b):
    return jax.lax.dot_general(a.astype(jnp.bfloat16), b.astype(jnp.bfloat16),
                               (((1,), (1,)), ((), ())),
                               preferred_element_type=jnp.float32)


def _pass1_body(x_ref, Lap_ref, W1_ref, b1_ref, W2_ref, b2_ref,
                Wg_ref, bg_ref, tg_ref, Wl_ref, bl_ref, tl_ref,
                yg_ref, yl_ref, h_ref, h16_ref):
    i = pl.program_id(0)

    @pl.when(i == 0)
    def _encode():
        h1 = jnp.maximum(_dotT(x_ref[...], W1_ref[...]) + b1_ref[...], 0.0)
        h2 = jnp.maximum(_dotT(h1, W2_ref[...]) + b2_ref[...], 0.0)
        h_ref[...] = h2
        h16_ref[...] = h2.astype(jnp.bfloat16)

    LX = _dot16(Lap_ref[...], h16_ref[...])
    rows = h_ref[pl.ds(i * RB, RB), :]
    zg = rows - tg_ref[0] * LX
    yg_ref[...] = jnp.maximum(_dotT(zg, Wg_ref[0]) + bg_ref[0], 0.0)
    zl = rows - tl_ref[0] * LX
    yl_ref[...] = jnp.maximum(_dotT(zl, Wl_ref[0]) + bl_ref[0], 0.0)


def _pass1(x, Lap, W1, b1, W2, b2, tg, Wg, bg, tl, Wl, bl):
    full = lambda *dims: pl.BlockSpec(dims, lambda i: (0,) * len(dims))
    return pl.pallas_call(
        _pass1_body,
        grid=(NBLK,),
        in_specs=[
            full(N, IN),
            pl.BlockSpec((RB, N), lambda i: (i, 0)),
            full(H1, IN), full(1, H1),
            full(H2, H1), full(1, H2),
            pl.BlockSpec((1, H2, H2), lambda i: (0, 0, 0)),
            pl.BlockSpec((1, 1, H2), lambda i: (0, 0, 0)),
            pl.BlockSpec(memory_space=pltpu.SMEM),
            pl.BlockSpec((1, H2, H2), lambda i: (0, 0, 0)),
            pl.BlockSpec((1, 1, H2), lambda i: (0, 0, 0)),
            pl.BlockSpec((1, 1, H2), lambda i: (0, 0, 0)),
        ],
        out_specs=[pl.BlockSpec((RB, H2), lambda i: (i, 0))] * 2,
        out_shape=[jax.ShapeDtypeStruct((N, H2), jnp.float32)] * 2,
        scratch_shapes=[pltpu.VMEM((N, H2), jnp.float32),
                        pltpu.VMEM((N, H2), jnp.bfloat16)],
    )(x, Lap, W1, b1.reshape(1, H1), W2, b2.reshape(1, H2),
      Wg, bg.reshape(2, 1, H2), tg, Wl, bl.reshape(2, 1, H2), tl.reshape(2, 1, H2))


def _pass2_body(Lap_ref, xg_ref, xl_ref,
                Wg_ref, bg_ref, tg_ref, Wl_ref, bl_ref, tl_ref,
                yg_ref, yl_ref, g16_ref, l16_ref):
    i = pl.program_id(0)

    @pl.when(i == 0)
    def _cast():
        g16_ref[...] = xg_ref[...].astype(jnp.bfloat16)
        l16_ref[...] = xl_ref[...].astype(jnp.bfloat16)

    Lap = Lap_ref[...].astype(jnp.bfloat16)
    LXg = _dot16(Lap, g16_ref[...])
    LXl = _dot16(Lap, l16_ref[...])
    zg = xg_ref[pl.ds(i * RB, RB), :] - tg_ref[1] * LXg
    yg_ref[...] = jnp.maximum(_dotT(zg, Wg_ref[0]) + bg_ref[0], 0.0)
    zl = xl_ref[pl.ds(i * RB, RB), :] - tl_ref[0] * LXl
    yl_ref[...] = jnp.maximum(_dotT(zl, Wl_ref[0]) + bl_ref[0], 0.0)


def _pass2(Lap, xg, xl, tg, Wg, bg, tl, Wl, bl):
    full = lambda *dims: pl.BlockSpec(dims, lambda i: (0,) * len(dims))
    return pl.pallas_call(
        _pass2_body,
        grid=(NBLK,),
        in_specs=[
            pl.BlockSpec((RB, N), lambda i: (i, 0)),
            full(N, H2), full(N, H2),
            pl.BlockSpec((1, H2, H2), lambda i: (1, 0, 0)),
            pl.BlockSpec((1, 1, H2), lambda i: (1, 0, 0)),
            pl.BlockSpec(memory_space=pltpu.SMEM),
            pl.BlockSpec((1, H2, H2), lambda i: (1, 0, 0)),
            pl.BlockSpec((1, 1, H2), lambda i: (1, 0, 0)),
            pl.BlockSpec((1, 1, H2), lambda i: (1, 0, 0)),
        ],
        out_specs=[pl.BlockSpec((RB, H2), lambda i: (i, 0))] * 2,
        out_shape=[jax.ShapeDtypeStruct((N, H2), jnp.float32)] * 2,
        scratch_shapes=[pltpu.VMEM((N, H2), jnp.bfloat16),
                        pltpu.VMEM((N, H2), jnp.bfloat16)],
    )(Lap, xg, xl, Wg, bg.reshape(2, 1, H2), tg, Wl, bl.reshape(2, 1, H2), tl.reshape(2, 1, H2))


def _loss_body(xg_ref, xl_ref, Wp1_ref, bp1_ref, Wp2_ref, bp2_ref,
               out_ref, A_ref, B_ref, c_ref, e_ref, acc_ref):
    i = pl.program_id(0)
    isq = 1.0 / (TAU ** 0.5)

    @pl.when(i == 0)
    def _project():
        zg = _dotT16(xg_ref[...], Wp1_ref[...]) + bp1_ref[...]
        ng = jnp.sqrt(jnp.sum(zg * zg, axis=1, keepdims=True))
        A_ref[...] = (zg * (isq / jnp.maximum(ng, 1e-12))).astype(jnp.bfloat16)
        zl = _dotT16(xl_ref[...], Wp2_ref[...]) + bp2_ref[...]
        nl = jnp.sqrt(jnp.sum(zl * zl, axis=1, keepdims=True))
        B_ref[...] = (zl * (isq / jnp.maximum(nl, 1e-12))).astype(jnp.bfloat16)
        acc_ref[0] = 0.0
        acc_ref[1] = 0.0

    A = A_ref[...]
    B = B_ref[...]
    A_blk = A_ref[pl.ds(i * RL, RL), :]
    B_blk = B_ref[pl.ds(i * RL, RL), :]
    Af = A_blk.astype(jnp.float32)
    Bf = B_blk.astype(jnp.float32)

    d_ab = jnp.sum(Af * Bf, axis=1, keepdims=True)
    e_ab = jnp.exp(d_ab)
    E_ab = jnp.exp(_dotT16(A_blk, B))
    r_ab = jnp.sum(E_ab, axis=1, keepdims=True) - e_ab
    c_ref[pl.ds(i, 1), :] = jnp.sum(E_ab, axis=0, keepdims=True)
    e_ref[pl.ds(i * RL, RL), :] = e_ab

    r_aa = (jnp.sum(jnp.exp(_dotT16(A_blk, A)), axis=1, keepdims=True)
            - jnp.exp(jnp.sum(Af * Af, axis=1, keepdims=True)))
    r_bb = (jnp.sum(jnp.exp(_dotT16(B_blk, B)), axis=1, keepdims=True)
            - jnp.exp(jnp.sum(Bf * Bf, axis=1, keepdims=True)))

    acc_ref[0] += jnp.sum(d_ab - jnp.log(r_ab) - jnp.log(r_aa))
    acc_ref[1] += jnp.sum(d_ab - jnp.log(r_bb))

    @pl.when(i == NL - 1)
    def _finish():
        csum = jnp.sum(c_ref[...], axis=0) - e_ref[...].reshape(N)
        loss1 = acc_ref[1] - jnp.sum(jnp.log(csum))
        out_ref[0, 0] = -(acc_ref[0] + loss1) / (2.0 * N)


def _loss(xg, xl, Wp1, bp1, Wp2, bp2):
    full = lambda *dims: pl.BlockSpec(dims, lambda i: (0,) * len(dims))
    return pl.pallas_call(
        _loss_body,
        grid=(NL,),
        in_specs=[
            full(N, H2), full(N, H2),
            full(H2, H2), full(1, H2),
            full(H2, H2), full(1, H2),
        ],
        out_specs=pl.BlockSpec(memory_space=pltpu.SMEM),
        out_shape=jax.ShapeDtypeStruct((1, 1), jnp.float32),
        scratch_shapes=[
            pltpu.VMEM((N, H2), jnp.bfloat16),
            pltpu.VMEM((N, H2), jnp.bfloat16),
            pltpu.VMEM((NL, N), jnp.float32),
            pltpu.VMEM((N, 1), jnp.float32),
            pltpu.SMEM((2,), jnp.float32),
        ],
    )(xg, xl, Wp1, bp1.reshape(1, H2), Wp2, bp2.reshape(1, H2))


def kernel(Lap, x, W1, b1, W2, b2, tg, Wg, bg, tl, Wl, bl, Wp1, bp1, Wp2, bp2):
    xg1, xl1 = _pass1(x, Lap, W1, b1, W2, b2, tg, Wg, bg, tl, Wl, bl)
    xg2, xl2 = _pass2(Lap, xg1, xl1, tg, Wg, bg, tl, Wl, bl)
    loss = _loss(xg2, xl2, Wp1, bp1, Wp2, bp2)
    return (xg2, xl2, loss[0, 0])


# speedup vs baseline: 2.9285x; 1.1010x over previous
import jax
import jax.numpy as jnp
from jax.experimental import pallas as pl
from jax.experimental.pallas import tpu as pltpu

N = 4096
IN = 256
H1 = 256
H2 = 128
TAU = 0.2

RB = 1024
NBLK = N // RB
RL = 512
NL = N // RL
STEPS = 2 * NBLK + NL


def _dotT(a, b):
    return jax.lax.dot_general(a, b, (((1,), (1,)), ((), ())),
                               preferred_element_type=jnp.float32)


def _dot16(a, b):
    return jax.lax.dot_general(a.astype(jnp.bfloat16), b.astype(jnp.bfloat16),
                               (((1,), (0,)), ((), ())),
                               preferred_element_type=jnp.float32)


def _dotT16(a, b):
    return jax.lax.dot_general(a.astype(jnp.bfloat16), b.astype(jnp.bfloat16),
                               (((1,), (1,)), ((), ())),
                               preferred_element_type=jnp.float32)


def _body(x_ref, Lap_ref, W1_ref, b1_ref, W2_ref, b2_ref,
          Wg_ref, bg_ref, tg_ref, Wl_ref, bl_ref, tl_ref,
          Wp1_ref, bp1_ref, Wp2_ref, bp2_ref,
          yg_ref, yl_ref, out_ref,
          h_ref, h16_ref, g1_ref, l1_ref, g16_ref, l16_ref,
          A_ref, B_ref, c_ref, e_ref, acc_ref):
    i = pl.program_id(0)

    @pl.when(i == 0)
    def _encode():
        h1 = jnp.maximum(_dotT(x_ref[...], W1_ref[...]) + b1_ref[...], 0.0)
        h2 = jnp.maximum(_dotT(h1, W2_ref[...]) + b2_ref[...], 0.0)
        h_ref[...] = h2
        h16_ref[...] = h2.astype(jnp.bfloat16)

    @pl.when(i < NBLK)
    def _diffuse0():
        LX = _dot16(Lap_ref[...], h16_ref[...])
        rows = h_ref[pl.ds(i * RB, RB), :]
        zg = rows - tg_ref[0] * LX
        yg = jnp.maximum(_dotT(zg, Wg_ref[0]) + bg_ref[0], 0.0)
        g1_ref[pl.ds(i * RB, RB), :] = yg
        g16_ref[pl.ds(i * RB, RB), :] = yg.astype(jnp.bfloat16)
        zl = rows - tl_ref[0] * LX
        yl = jnp.maximum(_dotT(zl, Wl_ref[0]) + bl_ref[0], 0.0)
        l1_ref[pl.ds(i * RB, RB), :] = yl
        l16_ref[pl.ds(i * RB, RB), :] = yl.astype(jnp.bfloat16)

    @pl.when((i >= NBLK) & (i < 2 * NBLK))
    def _diffuse1():
        j = i - NBLK
        Lap = Lap_ref[...].astype(jnp.bfloat16)
        LXg = _dot16(Lap, g16_ref[...])
        LXl = _dot16(Lap, l16_ref[...])
        zg = g1_ref[pl.ds(j * RB, RB), :] - tg_ref[1] * LXg
        yg = jnp.maximum(_dotT(zg, Wg_ref[1]) + bg_ref[1], 0.0)
        yg_ref[...] = yg
        zl = l1_ref[pl.ds(j * RB, RB), :] - tl_ref[1] * LXl
        yl = jnp.maximum(_dotT(zl, Wl_ref[1]) + bl_ref[1], 0.0)
        yl_ref[...] = yl
        A_ref[pl.ds(j * RB, RB), :] = yg.astype(jnp.bfloat16)
        B_ref[pl.ds(j * RB, RB), :] = yl.astype(jnp.bfloat16)

    isq = 1.0 / (TAU ** 0.5)

    @pl.when(i == 2 * NBLK)
    def _project():
        zg = _dotT16(A_ref[...], Wp1_ref[...]) + bp1_ref[...]
        ng = jnp.sqrt(jnp.sum(zg * zg, axis=1, keepdims=True))
        A_ref[...] = (zg * (isq / jnp.maximum(ng, 1e-12))).astype(jnp.bfloat16)
        zl = _dotT16(B_ref[...], Wp2_ref[...]) + bp2_ref[...]
        nl = jnp.sqrt(jnp.sum(zl * zl, axis=1, keepdims=True))
        B_ref[...] = (zl * (isq / jnp.maximum(nl, 1e-12))).astype(jnp.bfloat16)
        acc_ref[0] = 0.0
        acc_ref[1] = 0.0

    @pl.when(i >= 2 * NBLK)
    def _infonce_block():
        j = i - 2 * NBLK
        A = A_ref[...]
        B = B_ref[...]
        A_blk = A_ref[pl.ds(j * RL, RL), :]
        B_blk = B_ref[pl.ds(j * RL, RL), :]
        Af = A_blk.astype(jnp.float32)
        Bf = B_blk.astype(jnp.float32)

        d_ab = jnp.sum(Af * Bf, axis=1, keepdims=True)
        e_ab = jnp.exp(d_ab)
        E_ab = jnp.exp(_dotT16(A_blk, B))
        r_ab = jnp.sum(E_ab, axis=1, keepdims=True) - e_ab
        c_ref[pl.ds(j, 1), :] = jnp.sum(E_ab, axis=0, keepdims=True)
        e_ref[pl.ds(j * RL, RL), :] = e_ab

        r_aa = (jnp.sum(jnp.exp(_dotT16(A_blk, A)), axis=1, keepdims=True)
                - jnp.exp(jnp.sum(Af * Af, axis=1, keepdims=True)))
        r_bb = (jnp.sum(jnp.exp(_dotT16(B_blk, B)), axis=1, keepdims=True)
                - jnp.exp(jnp.sum(Bf * Bf, axis=1, keepdims=True)))

        acc_ref[0] += jnp.sum(d_ab - jnp.log(r_ab) - jnp.log(r_aa))
        acc_ref[1] += jnp.sum(d_ab - jnp.log(r_bb))

    @pl.when(i == STEPS - 1)
    def _finish():
        csum = jnp.sum(c_ref[...], axis=0) - e_ref[...].reshape(N)
        loss1 = acc_ref[1] - jnp.sum(jnp.log(csum))
        out_ref[0, 0] = -(acc_ref[0] + loss1) / (2.0 * N)


def kernel(Lap, x, W1, b1, W2, b2, tg, Wg, bg, tl, Wl, bl, Wp1, bp1, Wp2, bp2):
    full = lambda *dims: pl.BlockSpec(dims, lambda i: (0,) * len(dims))
    lap_idx = lambda i: (jnp.where(i < 2 * NBLK, i % NBLK, NBLK - 1), 0)
    out_idx = lambda i: (jnp.clip(i - NBLK, 0, NBLK - 1), 0)

    yg, yl, loss = pl.pallas_call(
        _body,
        grid=(STEPS,),
        in_specs=[
            full(N, IN),
            pl.BlockSpec((RB, N), lap_idx),
            full(H1, IN), full(1, H1),
            full(H2, H1), full(1, H2),
            full(2, H2, H2), full(2, 1, H2),
            pl.BlockSpec(memory_space=pltpu.SMEM),
            full(2, H2, H2), full(2, 1, H2),
            full(2, 1, H2),
            full(H2, H2), full(1, H2),
            full(H2, H2), full(1, H2),
        ],
        out_specs=[pl.BlockSpec((RB, H2), out_idx),
                   pl.BlockSpec((RB, H2), out_idx),
                   pl.BlockSpec(memory_space=pltpu.SMEM)],
        out_shape=[jax.ShapeDtypeStruct((N, H2), jnp.float32),
                   jax.ShapeDtypeStruct((N, H2), jnp.float32),
                   jax.ShapeDtypeStruct((1, 1), jnp.float32)],
        scratch_shapes=[
            pltpu.VMEM((N, H2), jnp.float32),
            pltpu.VMEM((N, H2), jnp.bfloat16),
            pltpu.VMEM((N, H2), jnp.float32),
            pltpu.VMEM((N, H2), jnp.float32),
            pltpu.VMEM((N, H2), jnp.bfloat16),
            pltpu.VMEM((N, H2), jnp.bfloat16),
            pltpu.VMEM((N, H2), jnp.bfloat16),
            pltpu.VMEM((N, H2), jnp.bfloat16),
            pltpu.VMEM((NL, N), jnp.float32),
            pltpu.VMEM((N, 1), jnp.float32),
            pltpu.SMEM((2,), jnp.float32),
        ],
    )(x, Lap, W1, b1.reshape(1, H1), W2, b2.reshape(1, H2),
      Wg, bg.reshape(2, 1, H2), tg, Wl, bl.reshape(2, 1, H2),
      tl.reshape(2, 1, H2), Wp1, bp1.reshape(1, H2), Wp2, bp2.reshape(1, H2))
    return (yg, yl, loss[0, 0])


# speedup vs baseline: 2.9443x; 1.0054x over previous
import jax
import jax.numpy as jnp
from jax.experimental import pallas as pl
from jax.experimental.pallas import tpu as pltpu

N = 4096
IN = 256
H1 = 256
H2 = 128
TAU = 0.2

RB = 1024
NBLK = N // RB
RL = 512
NL = N // RL
STEPS = 2 * NBLK + NL


def _dotT(a, b):
    return jax.lax.dot_general(a, b, (((1,), (1,)), ((), ())),
                               preferred_element_type=jnp.float32)


def _dot16(a, b):
    return jax.lax.dot_general(a.astype(jnp.bfloat16), b.astype(jnp.bfloat16),
                               (((1,), (0,)), ((), ())),
                               preferred_element_type=jnp.float32)


def _dotT16(a, b):
    return jax.lax.dot_general(a.astype(jnp.bfloat16), b.astype(jnp.bfloat16),
                               (((1,), (1,)), ((), ())),
                               preferred_element_type=jnp.float32)


def _body(x_ref, Lap_ref, W1_ref, b1_ref, W2_ref, b2_ref,
          Wg_ref, bg_ref, tg_ref, Wl_ref, bl_ref, tl_ref,
          Wp1_ref, bp1_ref, Wp2_ref, bp2_ref,
          yg_ref, yl_ref, out_ref,
          h_ref, h16_ref, g1_ref, l1_ref, g16_ref, l16_ref,
          A_ref, B_ref, c_ref, e_ref, acc_ref):
    i = pl.program_id(0)

    @pl.when(i == 0)
    def _encode():
        h1 = jnp.maximum(_dotT(x_ref[...], W1_ref[...]) + b1_ref[...], 0.0)
        h2 = jnp.maximum(_dotT(h1, W2_ref[...]) + b2_ref[...], 0.0)
        h_ref[...] = h2
        h16_ref[...] = h2.astype(jnp.bfloat16)

    @pl.when(i < NBLK)
    def _diffuse0():
        LX = _dot16(Lap_ref[...], h16_ref[...])
        rows = h_ref[pl.ds(i * RB, RB), :]
        zg = rows - tg_ref[0] * LX
        yg = jnp.maximum(_dotT(zg, Wg_ref[0]) + bg_ref[0], 0.0)
        g1_ref[pl.ds(i * RB, RB), :] = yg
        g16_ref[pl.ds(i * RB, RB), :] = yg.astype(jnp.bfloat16)
        zl = rows - tl_ref[0] * LX
        yl = jnp.maximum(_dotT(zl, Wl_ref[0]) + bl_ref[0], 0.0)
        l1_ref[pl.ds(i * RB, RB), :] = yl
        l16_ref[pl.ds(i * RB, RB), :] = yl.astype(jnp.bfloat16)

    @pl.when((i >= NBLK) & (i < 2 * NBLK))
    def _diffuse1():
        j = i - NBLK
        Lap = Lap_ref[...].astype(jnp.bfloat16)
        LXg = _dot16(Lap, g16_ref[...])
        LXl = _dot16(Lap, l16_ref[...])
        zg = g1_ref[pl.ds(j * RB, RB), :] - tg_ref[1] * LXg
        yg = jnp.maximum(_dotT(zg, Wg_ref[1]) + bg_ref[1], 0.0)
        yg_ref[...] = yg
        zl = l1_ref[pl.ds(j * RB, RB), :] - tl_ref[1] * LXl
        yl = jnp.maximum(_dotT(zl, Wl_ref[1]) + bl_ref[1], 0.0)
        yl_ref[...] = yl
        A_ref[pl.ds(j * RB, RB), :] = yg.astype(jnp.bfloat16)
        B_ref[pl.ds(j * RB, RB), :] = yl.astype(jnp.bfloat16)

    import math
    isq = (math.log2(math.e) / TAU) ** 0.5
    ln2 = math.log(2.0)

    @pl.when(i == 2 * NBLK)
    def _project():
        zg = _dotT16(A_ref[...], Wp1_ref[...]) + bp1_ref[...]
        ng = jnp.sqrt(jnp.sum(zg * zg, axis=1, keepdims=True))
        A_ref[...] = (zg * (isq / jnp.maximum(ng, 1e-12))).astype(jnp.bfloat16)
        zl = _dotT16(B_ref[...], Wp2_ref[...]) + bp2_ref[...]
        nl = jnp.sqrt(jnp.sum(zl * zl, axis=1, keepdims=True))
        B_ref[...] = (zl * (isq / jnp.maximum(nl, 1e-12))).astype(jnp.bfloat16)
        acc_ref[0] = 0.0
        acc_ref[1] = 0.0

    @pl.when(i >= 2 * NBLK)
    def _infonce_block():
        j = i - 2 * NBLK
        A = A_ref[...]
        B = B_ref[...]
        A_blk = A_ref[pl.ds(j * RL, RL), :]
        B_blk = B_ref[pl.ds(j * RL, RL), :]
        Af = A_blk.astype(jnp.float32)
        Bf = B_blk.astype(jnp.float32)

        d2_ab = jnp.sum(Af * Bf, axis=1, keepdims=True)
        d_ab = d2_ab * ln2
        e_ab = jnp.exp2(d2_ab)
        E_ab = jnp.exp2(_dotT16(A_blk, B))
        r_ab = jnp.sum(E_ab, axis=1, keepdims=True) - e_ab
        c_ref[pl.ds(j, 1), :] = jnp.sum(E_ab, axis=0, keepdims=True)
        e_ref[pl.ds(j * RL, RL), :] = e_ab

        r_aa = (jnp.sum(jnp.exp2(_dotT16(A_blk, A)), axis=1, keepdims=True)
                - jnp.exp2(jnp.sum(Af * Af, axis=1, keepdims=True)))
        r_bb = (jnp.sum(jnp.exp2(_dotT16(B_blk, B)), axis=1, keepdims=True)
                - jnp.exp2(jnp.sum(Bf * Bf, axis=1, keepdims=True)))

        acc_ref[0] += jnp.sum(d_ab - jnp.log(r_ab) - jnp.log(r_aa))
        acc_ref[1] += jnp.sum(d_ab - jnp.log(r_bb))

    @pl.when(i == STEPS - 1)
    def _finish():
        csum = jnp.sum(c_ref[...], axis=0) - e_ref[...].reshape(N)
        loss1 = acc_ref[1] - jnp.sum(jnp.log(csum))
        out_ref[0, 0] = -(acc_ref[0] + loss1) / (2.0 * N)


def kernel(Lap, x, W1, b1, W2, b2, tg, Wg, bg, tl, Wl, bl, Wp1, bp1, Wp2, bp2):
    full = lambda *dims: pl.BlockSpec(dims, lambda i: (0,) * len(dims))
    lap_idx = lambda i: (jnp.where(i < 2 * NBLK, i % NBLK, NBLK - 1), 0)
    out_idx = lambda i: (jnp.clip(i - NBLK, 0, NBLK - 1), 0)

    yg, yl, loss = pl.pallas_call(
        _body,
        grid=(STEPS,),
        in_specs=[
            full(N, IN),
            pl.BlockSpec((RB, N), lap_idx),
            full(H1, IN), full(1, H1),
            full(H2, H1), full(1, H2),
            full(2, H2, H2), full(2, 1, H2),
            pl.BlockSpec(memory_space=pltpu.SMEM),
            full(2, H2, H2), full(2, 1, H2),
            full(2, 1, H2),
            full(H2, H2), full(1, H2),
            full(H2, H2), full(1, H2),
        ],
        out_specs=[pl.BlockSpec((RB, H2), out_idx),
                   pl.BlockSpec((RB, H2), out_idx),
                   pl.BlockSpec(memory_space=pltpu.SMEM)],
        out_shape=[jax.ShapeDtypeStruct((N, H2), jnp.float32),
                   jax.ShapeDtypeStruct((N, H2), jnp.float32),
                   jax.ShapeDtypeStruct((1, 1), jnp.float32)],
        scratch_shapes=[
            pltpu.VMEM((N, H2), jnp.float32),
            pltpu.VMEM((N, H2), jnp.bfloat16),
            pltpu.VMEM((N, H2), jnp.float32),
            pltpu.VMEM((N, H2), jnp.float32),
            pltpu.VMEM((N, H2), jnp.bfloat16),
            pltpu.VMEM((N, H2), jnp.bfloat16),
            pltpu.VMEM((N, H2), jnp.bfloat16),
            pltpu.VMEM((N, H2), jnp.bfloat16),
            pltpu.VMEM((NL, N), jnp.float32),
            pltpu.VMEM((N, 1), jnp.float32),
            pltpu.SMEM((2,), jnp.float32),
        ],
    )(x, Lap, W1, b1.reshape(1, H1), W2, b2.reshape(1, H2),
      Wg, bg.reshape(2, 1, H2), tg, Wl, bl.reshape(2, 1, H2),
      tl.reshape(2, 1, H2), Wp1, bp1.reshape(1, H2), Wp2, bp2.reshape(1, H2))
    return (yg, yl, loss[0, 0])


# speedup vs baseline: 2.9981x; 1.0183x over previous
import math

import jax
import jax.numpy as jnp
from jax.experimental import pallas as pl
from jax.experimental.pallas import tpu as pltpu

N = 4096
IN = 256
H1 = 256
H2 = 128
TAU = 0.2

RB = 1024
NBLK = N // RB
RL = 512
NL = N // RL
STEPS = 2 * NBLK + NL


def _dotT(a, b):
    return jax.lax.dot_general(a, b, (((1,), (1,)), ((), ())),
                               preferred_element_type=jnp.float32)


def _dot16(a, b):
    return jax.lax.dot_general(a.astype(jnp.bfloat16), b.astype(jnp.bfloat16),
                               (((1,), (0,)), ((), ())),
                               preferred_element_type=jnp.float32)


def _dotT16(a, b):
    return jax.lax.dot_general(a.astype(jnp.bfloat16), b.astype(jnp.bfloat16),
                               (((1,), (1,)), ((), ())),
                               preferred_element_type=jnp.float32)


def _body(x_ref, Lap_ref, W1_ref, b1_ref, W2_ref, b2_ref,
          Wg_ref, bg_ref, tg_ref, Wl_ref, bl_ref, tl_ref,
          Wp1_ref, bp1_ref, Wp2_ref, bp2_ref,
          yg_ref, yl_ref, out_ref,
          h_ref, h16_ref, g1_ref, l1_ref, g16_ref, l16_ref,
          A_ref, B_ref, c_ref, e_ref, acc_ref):
    i = pl.program_id(0)
    isq = (math.log2(math.e) / TAU) ** 0.5
    ln2 = math.log(2.0)

    @pl.when(i == 0)
    def _encode():
        acc_ref[0] = 0.0
        acc_ref[1] = 0.0
        h1 = jnp.maximum(_dotT(x_ref[...], W1_ref[...]) + b1_ref[...], 0.0)
        h2 = jnp.maximum(_dotT(h1, W2_ref[...]) + b2_ref[...], 0.0)
        h_ref[...] = h2
        h16_ref[...] = h2.astype(jnp.bfloat16)

    @pl.when(i < NBLK)
    def _diffuse0():
        LX = _dot16(Lap_ref[...], h16_ref[...])
        rows = h_ref[pl.ds(i * RB, RB), :]
        zg = rows - tg_ref[0] * LX
        yg = jnp.maximum(_dotT(zg, Wg_ref[0]) + bg_ref[0], 0.0)
        g1_ref[pl.ds(i * RB, RB), :] = yg
        g16_ref[pl.ds(i * RB, RB), :] = yg.astype(jnp.bfloat16)
        zl = rows - tl_ref[0] * LX
        yl = jnp.maximum(_dotT(zl, Wl_ref[0]) + bl_ref[0], 0.0)
        l1_ref[pl.ds(i * RB, RB), :] = yl
        l16_ref[pl.ds(i * RB, RB), :] = yl.astype(jnp.bfloat16)

    @pl.when((i >= NBLK) & (i < 2 * NBLK))
    def _diffuse1():
        j = i - NBLK
        Lap = Lap_ref[...].astype(jnp.bfloat16)
        LXg = _dot16(Lap, g16_ref[...])
        LXl = _dot16(Lap, l16_ref[...])
        zg = g1_ref[pl.ds(j * RB, RB), :] - tg_ref[1] * LXg
        yg = jnp.maximum(_dotT(zg, Wg_ref[1]) + bg_ref[1], 0.0)
        yg_ref[...] = yg
        zl = l1_ref[pl.ds(j * RB, RB), :] - tl_ref[1] * LXl
        yl = jnp.maximum(_dotT(zl, Wl_ref[1]) + bl_ref[1], 0.0)
        yl_ref[...] = yl
        zpg = _dotT16(yg, Wp1_ref[...]) + bp1_ref[...]
        n2g = jnp.sum(zpg * zpg, axis=1, keepdims=True)
        A_ref[pl.ds(j * RB, RB), :] = (
            zpg * (isq * jax.lax.rsqrt(jnp.maximum(n2g, 1e-24)))
        ).astype(jnp.bfloat16)
        zpl = _dotT16(yl, Wp2_ref[...]) + bp2_ref[...]
        n2l = jnp.sum(zpl * zpl, axis=1, keepdims=True)
        B_ref[pl.ds(j * RB, RB), :] = (
            zpl * (isq * jax.lax.rsqrt(jnp.maximum(n2l, 1e-24)))
        ).astype(jnp.bfloat16)

    @pl.when(i >= 2 * NBLK)
    def _infonce_block():
        j = i - 2 * NBLK
        A = A_ref[...]
        B = B_ref[...]
        A_blk = A_ref[pl.ds(j * RL, RL), :]
        B_blk = B_ref[pl.ds(j * RL, RL), :]
        Af = A_blk.astype(jnp.float32)
        Bf = B_blk.astype(jnp.float32)

        d2_ab = jnp.sum(Af * Bf, axis=1, keepdims=True)
        d_ab = d2_ab * ln2
        e_ab = jnp.exp2(d2_ab)
        E_ab = jnp.exp2(_dotT16(A_blk, B))
        r_ab = jnp.sum(E_ab, axis=1, keepdims=True) - e_ab
        c_ref[pl.ds(j, 1), :] = jnp.sum(E_ab, axis=0, keepdims=True)
        e_ref[pl.ds(j * RL, RL), :] = e_ab

        r_aa = (jnp.sum(jnp.exp2(_dotT16(A_blk, A)), axis=1, keepdims=True)
                - jnp.exp2(jnp.sum(Af * Af, axis=1, keepdims=True)))
        r_bb = (jnp.sum(jnp.exp2(_dotT16(B_blk, B)), axis=1, keepdims=True)
                - jnp.exp2(jnp.sum(Bf * Bf, axis=1, keepdims=True)))

        acc_ref[0] += jnp.sum(d_ab - jnp.log(r_ab) - jnp.log(r_aa))
        acc_ref[1] += jnp.sum(d_ab - jnp.log(r_bb))

    @pl.when(i == STEPS - 1)
    def _finish():
        csum = jnp.sum(c_ref[...], axis=0) - e_ref[...].reshape(N)
        loss1 = acc_ref[1] - jnp.sum(jnp.log(csum))
        out_ref[0, 0] = -(acc_ref[0] + loss1) / (2.0 * N)


def kernel(Lap, x, W1, b1, W2, b2, tg, Wg, bg, tl, Wl, bl, Wp1, bp1, Wp2, bp2):
    full = lambda *dims: pl.BlockSpec(dims, lambda i: (0,) * len(dims))
    lap_idx = lambda i: (jnp.where(i < 2 * NBLK, i % NBLK, NBLK - 1), 0)
    out_idx = lambda i: (jnp.clip(i - NBLK, 0, NBLK - 1), 0)

    yg, yl, loss = pl.pallas_call(
        _body,
        grid=(STEPS,),
        in_specs=[
            full(N, IN),
            pl.BlockSpec((RB, N), lap_idx),
            full(H1, IN), full(1, H1),
            full(H2, H1), full(1, H2),
            full(2, H2, H2), full(2, 1, H2),
            pl.BlockSpec(memory_space=pltpu.SMEM),
            full(2, H2, H2), full(2, 1, H2),
            full(2, 1, H2),
            full(H2, H2), full(1, H2),
            full(H2, H2), full(1, H2),
        ],
        out_specs=[pl.BlockSpec((RB, H2), out_idx),
                   pl.BlockSpec((RB, H2), out_idx),
                   pl.BlockSpec(memory_space=pltpu.SMEM)],
        out_shape=[jax.ShapeDtypeStruct((N, H2), jnp.float32),
                   jax.ShapeDtypeStruct((N, H2), jnp.float32),
                   jax.ShapeDtypeStruct((1, 1), jnp.float32)],
        scratch_shapes=[
            pltpu.VMEM((N, H2), jnp.float32),
            pltpu.VMEM((N, H2), jnp.bfloat16),
            pltpu.VMEM((N, H2), jnp.float32),
            pltpu.VMEM((N, H2), jnp.float32),
            pltpu.VMEM((N, H2), jnp.bfloat16),
            pltpu.VMEM((N, H2), jnp.bfloat16),
            pltpu.VMEM((N, H2), jnp.bfloat16),
            pltpu.VMEM((N, H2), jnp.bfloat16),
            pltpu.VMEM((NL, N), jnp.float32),
            pltpu.VMEM((N, 1), jnp.float32),
            pltpu.SMEM((2,), jnp.float32),
        ],
    )(x, Lap, W1, b1.reshape(1, H1), W2, b2.reshape(1, H2),
      Wg, bg.reshape(2, 1, H2), tg, Wl, bl.reshape(2, 1, H2),
      tl.reshape(2, 1, H2), Wp1, bp1.reshape(1, H2), Wp2, bp2.reshape(1, H2))
    return (yg, yl, loss[0, 0])


# speedup vs baseline: 3.1388x; 1.0469x over previous
import math

import jax
import jax.numpy as jnp
from jax.experimental import pallas as pl
from jax.experimental.pallas import tpu as pltpu

N = 4096
IN = 256
H1 = 256
H2 = 128
TAU = 0.2

RB = 1024
NBLK = N // RB
RL = 512
NL = N // RL
STEPS = 2 * NBLK + NL


def _dotT(a, b):
    return jax.lax.dot_general(a, b, (((1,), (1,)), ((), ())),
                               preferred_element_type=jnp.float32)


def _dot16(a, b):
    return jax.lax.dot_general(a.astype(jnp.bfloat16), b.astype(jnp.bfloat16),
                               (((1,), (0,)), ((), ())),
                               preferred_element_type=jnp.float32)


def _dotT16(a, b):
    return jax.lax.dot_general(a.astype(jnp.bfloat16), b.astype(jnp.bfloat16),
                               (((1,), (1,)), ((), ())),
                               preferred_element_type=jnp.float32)


def _body(x_ref, Lap_ref, W1_ref, b1_ref, W2_ref, b2_ref,
          Wg_ref, bg_ref, tg_ref, Wl_ref, bl_ref, tl_ref,
          Wp1_ref, bp1_ref, Wp2_ref, bp2_ref,
          yg_ref, yl_ref, out_ref,
          h_ref, h16_ref, g1_ref, l1_ref, g16_ref, l16_ref,
          A_ref, B_ref, c_ref, e_ref, raa_ref, rbb_ref, ua_ref, ub_ref,
          acc_ref):
    i = pl.program_id(0)
    isq = (math.log2(math.e) / TAU) ** 0.5
    ln2 = math.log(2.0)

    @pl.when(i == 0)
    def _encode():
        acc_ref[0] = 0.0
        acc_ref[1] = 0.0
        ua_ref[...] = jnp.zeros((NL, N), jnp.float32)
        ub_ref[...] = jnp.zeros((NL, N), jnp.float32)
        h1 = jnp.maximum(_dotT(x_ref[...], W1_ref[...]) + b1_ref[...], 0.0)
        h2 = jnp.maximum(_dotT(h1, W2_ref[...]) + b2_ref[...], 0.0)
        h_ref[...] = h2
        h16_ref[...] = h2.astype(jnp.bfloat16)

    @pl.when(i < NBLK)
    def _diffuse0():
        LX = _dot16(Lap_ref[...], h16_ref[...])
        rows = h_ref[pl.ds(i * RB, RB), :]
        zg = rows - tg_ref[0] * LX
        yg = jnp.maximum(_dotT(zg, Wg_ref[0]) + bg_ref[0], 0.0)
        g1_ref[pl.ds(i * RB, RB), :] = yg
        g16_ref[pl.ds(i * RB, RB), :] = yg.astype(jnp.bfloat16)
        zl = rows - tl_ref[0] * LX
        yl = jnp.maximum(_dotT(zl, Wl_ref[0]) + bl_ref[0], 0.0)
        l1_ref[pl.ds(i * RB, RB), :] = yl
        l16_ref[pl.ds(i * RB, RB), :] = yl.astype(jnp.bfloat16)

    @pl.when((i >= NBLK) & (i < 2 * NBLK))
    def _diffuse1():
        j = i - NBLK
        Lap = Lap_ref[...].astype(jnp.bfloat16)
        LXg = _dot16(Lap, g16_ref[...])
        LXl = _dot16(Lap, l16_ref[...])
        zg = g1_ref[pl.ds(j * RB, RB), :] - tg_ref[1] * LXg
        yg = jnp.maximum(_dotT(zg, Wg_ref[1]) + bg_ref[1], 0.0)
        yg_ref[...] = yg
        zl = l1_ref[pl.ds(j * RB, RB), :] - tl_ref[1] * LXl
        yl = jnp.maximum(_dotT(zl, Wl_ref[1]) + bl_ref[1], 0.0)
        yl_ref[...] = yl
        zpg = _dotT16(yg, Wp1_ref[...]) + bp1_ref[...]
        n2g = jnp.sum(zpg * zpg, axis=1, keepdims=True)
        A_ref[pl.ds(j * RB, RB), :] = (
            zpg * (isq * jax.lax.rsqrt(jnp.maximum(n2g, 1e-24)))
        ).astype(jnp.bfloat16)
        zpl = _dotT16(yl, Wp2_ref[...]) + bp2_ref[...]
        n2l = jnp.sum(zpl * zpl, axis=1, keepdims=True)
        B_ref[pl.ds(j * RB, RB), :] = (
            zpl * (isq * jax.lax.rsqrt(jnp.maximum(n2l, 1e-24)))
        ).astype(jnp.bfloat16)

    for jj in range(NL):
        @pl.when(i == 2 * NBLK + jj)
        def _infonce_block(jj=jj):
            lo = jj * RL
            W = N - lo
            A_blk = A_ref[pl.ds(lo, RL), :]
            B_blk = B_ref[pl.ds(lo, RL), :]
            Af = A_blk.astype(jnp.float32)
            Bf = B_blk.astype(jnp.float32)

            d2_ab = jnp.sum(Af * Bf, axis=1, keepdims=True)
            e_ab = jnp.exp2(d2_ab)
            E_ab = jnp.exp2(_dotT16(A_blk, B_ref[...]))
            r_ab = jnp.sum(E_ab, axis=1, keepdims=True) - e_ab
            c_ref[pl.ds(jj, 1), :] = jnp.sum(E_ab, axis=0, keepdims=True)
            e_ref[pl.ds(lo, RL), :] = e_ab
            acc_ref[0] += jnp.sum(d2_ab * ln2 - jnp.log(r_ab))
            acc_ref[1] += jnp.sum(d2_ab * ln2)

            F_aa = jnp.exp2(_dotT16(A_blk, A_ref[pl.ds(lo, W), :]))
            raa_ref[pl.ds(lo, RL), :] = (
                jnp.sum(F_aa, axis=1, keepdims=True)
                - jnp.exp2(jnp.sum(Af * Af, axis=1, keepdims=True)))
            F_bb = jnp.exp2(_dotT16(B_blk, B_ref[pl.ds(lo, W), :]))
            rbb_ref[pl.ds(lo, RL), :] = (
                jnp.sum(F_bb, axis=1, keepdims=True)
                - jnp.exp2(jnp.sum(Bf * Bf, axis=1, keepdims=True)))
            if jj < NL - 1:
                ua_ref[pl.ds(jj, 1), pl.ds(lo + RL, W - RL)] = (
                    jnp.sum(F_aa[:, RL:], axis=0, keepdims=True))
                ub_ref[pl.ds(jj, 1), pl.ds(lo + RL, W - RL)] = (
                    jnp.sum(F_bb[:, RL:], axis=0, keepdims=True))

    @pl.when(i == STEPS - 1)
    def _finish():
        raa = raa_ref[...].reshape(N) + jnp.sum(ua_ref[...], axis=0)
        rbb = rbb_ref[...].reshape(N) + jnp.sum(ub_ref[...], axis=0)
        csum = jnp.sum(c_ref[...], axis=0) - e_ref[...].reshape(N)
        loss0 = acc_ref[0] - jnp.sum(jnp.log(raa))
        loss1 = acc_ref[1] - jnp.sum(jnp.log(csum)) - jnp.sum(jnp.log(rbb))
        out_ref[0, 0] = -(loss0 + loss1) / (2.0 * N)


def kernel(Lap, x, W1, b1, W2, b2, tg, Wg, bg, tl, Wl, bl, Wp1, bp1, Wp2, bp2):
    full = lambda *dims: pl.BlockSpec(dims, lambda i: (0,) * len(dims))
    lap_idx = lambda i: (jnp.where(i < 2 * NBLK, i % NBLK, NBLK - 1), 0)
    out_idx = lambda i: (jnp.clip(i - NBLK, 0, NBLK - 1), 0)

    yg, yl, loss = pl.pallas_call(
        _body,
        grid=(STEPS,),
        compiler_params=pltpu.CompilerParams(
            vmem_limit_bytes=100 * 1024 * 1024),
        in_specs=[
            full(N, IN),
            pl.BlockSpec((RB, N), lap_idx),
            full(H1, IN), full(1, H1),
            full(H2, H1), full(1, H2),
            full(2, H2, H2), full(2, 1, H2),
            pl.BlockSpec(memory_space=pltpu.SMEM),
            full(2, H2, H2), full(2, 1, H2),
            full(2, 1, H2),
            full(H2, H2), full(1, H2),
            full(H2, H2), full(1, H2),
        ],
        out_specs=[pl.BlockSpec((RB, H2), out_idx),
                   pl.BlockSpec((RB, H2), out_idx),
                   pl.BlockSpec(memory_space=pltpu.SMEM)],
        out_shape=[jax.ShapeDtypeStruct((N, H2), jnp.float32),
                   jax.ShapeDtypeStruct((N, H2), jnp.float32),
                   jax.ShapeDtypeStruct((1, 1), jnp.float32)],
        scratch_shapes=[
            pltpu.VMEM((N, H2), jnp.float32),
            pltpu.VMEM((N, H2), jnp.bfloat16),
            pltpu.VMEM((N, H2), jnp.float32),
            pltpu.VMEM((N, H2), jnp.float32),
            pltpu.VMEM((N, H2), jnp.bfloat16),
            pltpu.VMEM((N, H2), jnp.bfloat16),
            pltpu.VMEM((N, H2), jnp.bfloat16),
            pltpu.VMEM((N, H2), jnp.bfloat16),
            pltpu.VMEM((NL, N), jnp.float32),
            pltpu.VMEM((N, 1), jnp.float32),
            pltpu.VMEM((N, 1), jnp.float32),
            pltpu.VMEM((N, 1), jnp.float32),
            pltpu.VMEM((NL, N), jnp.float32),
            pltpu.VMEM((NL, N), jnp.float32),
            pltpu.SMEM((2,), jnp.float32),
        ],
    )(x, Lap, W1, b1.reshape(1, H1), W2, b2.reshape(1, H2),
      Wg, bg.reshape(2, 1, H2), tg, Wl, bl.reshape(2, 1, H2),
      tl.reshape(2, 1, H2), Wp1, bp1.reshape(1, H2), Wp2, bp2.reshape(1, H2))
    return (yg, yl, loss[0, 0])
